# baseline (speedup 1.0000x reference)
import functools

import jax
import jax.numpy as jnp
from jax import lax
from jax.experimental import pallas as pl
from jax.experimental.pallas import tpu as pltpu
from jax.experimental.pallas import tpu_sc as plsc

_THRESHOLDS = (1, 2, 3, 4, 5, 6, 7, 8, 12, 16, 23, 32, 46, 64, 91)

_N_HEADS = 16
_Q = 2048
_K = 2048
_DV = 4096
_DA = 3968
_N_SHIFTS = 128
_HPC = _N_HEADS // 2
_NBUF = 3
_LANES = 16
_RPS = _N_SHIFTS // 16


def _diag_body(delta_ref, embt_ref, out_ref):
    dd = delta_ref[0]
    xg = lax.broadcasted_iota(jnp.int32, (32, _DV), 1)
    bb = lax.broadcasted_iota(jnp.int32, (32, _DV), 0)
    rp = xg - (_Q - 1) + dd
    a = jnp.abs(rp)
    g = jnp.zeros((32, _DV), jnp.int32)
    for t in _THRESHOLDS:
        g = g + (a >= t).astype(jnp.int32)
    bucket = jnp.where(rp > 0, 16, 0) + g
    onehot = (bucket == bb).astype(jnp.float32)
    hh = pl.program_id(0)
    row = lax.dot_general(
        embt_ref[pl.ds(hh, 1), :], onehot,
        dimension_numbers=(((1,), (0,)), ((), ())),
        preferred_element_type=jnp.float32,
        precision=lax.Precision.HIGHEST,
    )
    out_ref[...] = row[0, :]


def _build_diag(delta, emb_t):
    return pl.pallas_call(
        _diag_body,
        grid=(_N_HEADS,),
        out_shape=jax.ShapeDtypeStruct((_N_HEADS * _DV,), jnp.float32),
        in_specs=[
            pl.BlockSpec(memory_space=pltpu.SMEM),
            pl.BlockSpec((16, 32), lambda h: (0, 0)),
        ],
        out_specs=pl.BlockSpec((_DV,), lambda h: (h,)),
    )(delta, emb_t)


def _materialize_body(diag_hbm, out_hbm, *refs):
    spms = refs[:_NBUF]
    diagpad = refs[_NBUF]
    chunk0, chunk1 = refs[_NBUF + 1], refs[_NBUF + 2]
    sem_c0, sem_c1, sem_w = refs[_NBUF + 3], refs[_NBUF + 4], refs[_NBUF + 5]
    c = lax.axis_index("c")
    sid = lax.axis_index("s")
    h0 = c * _HPC
    start = pl.multiple_of((_Q - 128) - 128 * sid, 128)
    row0 = pl.multiple_of(128 * sid, 8)
    srow = pl.multiple_of(_RPS * sid, 8)
    chunks = (chunk0, chunk1)
    sems_c = (sem_c0, sem_c1)
    n_ck = _DA // 128

    pltpu.sync_copy(diag_hbm.at[pl.ds(c * (_HPC * _DV), _HPC * _DV)], diagpad)

    def build(idx):
        base = idx * _DV + 127 - _RPS * sid
        spm = spms[idx % _NBUF]

        def one_chunk(ck, q, first):
            col0 = pl.multiple_of(128 * ck, 128)
            cdesc = pltpu.make_async_copy(
                chunks[q],
                spm.at[pl.ds(srow, _RPS), pl.ds(col0, 128)],
                sems_c[q],
            )
            if first is None:
                cdesc.wait()
            elif first is not True:
                @pl.when(first)
                def _wait_prev():
                    cdesc.wait()

            for w in range(128 // _LANES):
                col = col0 + _LANES * w
                for r in range(_RPS):
                    chunks[q][r, pl.ds(_LANES * w, _LANES)] = diagpad[
                        pl.ds(base - r + col, _LANES)
                    ]
            cdesc.start()

        def pair(kk, carry):
            one_chunk(2 * kk, 0, kk >= 1)
            one_chunk(2 * kk + 1, 1, kk >= 1)
            return carry

        lax.fori_loop(0, n_ck // 2, pair, jnp.int32(0))
        one_chunk(n_ck - 1, 0, None)
        pltpu.make_async_copy(
            chunks[0], spm.at[pl.ds(srow, _RPS), pl.ds(0, 128)], sems_c[0]
        ).wait()
        pltpu.make_async_copy(
            chunks[1], spm.at[pl.ds(srow, _RPS), pl.ds(0, 128)], sems_c[1]
        ).wait()

    def write(idx):
        return pltpu.make_async_copy(
            spms[idx % _NBUF].at[:, pl.ds(start, _K)],
            out_hbm.at[h0 + idx, pl.ds(row0, 128), :],
            sem_w,
        )

    build(0)

    for idx in range(_HPC):
        if idx >= 2:
            write(idx - 2).wait()
        plsc.subcore_barrier()
        write(idx).start()
        if idx + 1 < _HPC:
            build(idx + 1)

    write(_HPC - 2).wait()
    write(_HPC - 1).wait()


@functools.cache
def _make_materialize():
    mesh = plsc.VectorSubcoreMesh(core_axis_name="c", subcore_axis_name="s")
    return pl.kernel(
        _materialize_body,
        mesh=mesh,
        out_type=jax.ShapeDtypeStruct((_N_HEADS, _Q, _K), jnp.float32),
        scratch_types=(
            [pltpu.VMEM_SHARED((_N_SHIFTS, _DA), jnp.float32)] * _NBUF
            + [
                pltpu.VMEM((_HPC * _DV,), jnp.float32),
                pltpu.VMEM((_RPS, 128), jnp.float32),
                pltpu.VMEM((_RPS, 128), jnp.float32),
                pltpu.SemaphoreType.DMA,
                pltpu.SemaphoreType.DMA,
                pltpu.SemaphoreType.DMA,
            ]
        ),
    )


def kernel(query_length, key_length, offset, embeddings):
    del offset
    delta = (
        jnp.asarray(key_length, jnp.int32) - jnp.asarray(query_length, jnp.int32)
    ).reshape(1)
    diag = _build_diag(delta, embeddings.T)
    return _make_materialize()(diag)


# speedup vs baseline: 1.0215x; 1.0215x over previous
import functools

import jax
import jax.numpy as jnp
from jax import lax
from jax.experimental import pallas as pl
from jax.experimental.pallas import tpu as pltpu
from jax.experimental.pallas import tpu_sc as plsc

_THRESHOLDS = (1, 2, 3, 4, 5, 6, 7, 8, 12, 16, 23, 32, 46, 64, 91)

_N_HEADS = 16
_Q = 2048
_K = 2048
_DV = 4096
_DA = 3968
_N_SHIFTS = 128
_HPC = _N_HEADS // 2
_NBUF = 4


def _diag_body(delta_ref, emb_ref, out_ref):
    dd = delta_ref[0]
    xg = lax.broadcasted_iota(jnp.int32, (32, _DV), 1)
    bb = lax.broadcasted_iota(jnp.int32, (32, _DV), 0)
    rp = xg - (_Q - 1) + dd
    a = jnp.abs(rp)
    g = jnp.zeros((32, _DV), jnp.int32)
    for t in _THRESHOLDS:
        g = g + (a >= t).astype(jnp.int32)
    bucket = jnp.where(rp > 0, 16, 0) + g
    onehot = (bucket == bb).astype(jnp.float32)
    hh = pl.program_id(0)
    row = lax.dot_general(
        emb_ref[pl.ds(hh, 1), :], onehot,
        dimension_numbers=(((1,), (0,)), ((), ())),
        preferred_element_type=jnp.float32,
        precision=lax.Precision.HIGHEST,
    )
    for s in range(_N_SHIFTS):
        out_ref[0, s, :] = row[0, 127 - s : 127 - s + _DA]


def _build_shifts(delta, emb):
    return pl.pallas_call(
        _diag_body,
        grid=(_N_HEADS,),
        out_shape=jax.ShapeDtypeStruct(
            (_N_HEADS, _N_SHIFTS, _DA), jnp.float32
        ),
        in_specs=[
            pl.BlockSpec(memory_space=pltpu.SMEM),
            pl.BlockSpec((16, 32), lambda h: (0, 0)),
        ],
        out_specs=pl.BlockSpec((1, _N_SHIFTS, _DA), lambda h: (h, 0, 0)),
    )(delta, emb)


def _materialize_body(shifts_hbm, out_hbm, *refs):
    spms = refs[:_NBUF]
    sems = refs[_NBUF:2 * _NBUF]
    sem_w = refs[2 * _NBUF]
    c = lax.axis_index("c")
    sid = lax.axis_index("s")
    h0 = c * _HPC
    start = pl.multiple_of((_Q - 128) - 128 * sid, 128)
    row0 = pl.multiple_of(128 * sid, 8)

    def stage(idx):
        return pltpu.make_async_copy(
            shifts_hbm.at[h0 + idx], spms[idx % _NBUF], sems[idx % _NBUF]
        )

    def write(idx):
        return pltpu.make_async_copy(
            spms[idx % _NBUF].at[:, pl.ds(start, _K)],
            out_hbm.at[h0 + idx, pl.ds(row0, 128), :],
            sem_w,
        )

    @pl.when(sid == 0)
    def _prologue():
        stage(0).start()
        stage(1).start()

    for idx in range(_HPC):
        if idx >= _NBUF - 2:
            write(idx - _NBUF + 2).wait()
        plsc.subcore_barrier()

        @pl.when(sid == 0)
        def _stager(idx=idx):
            if idx + 2 < _HPC:
                stage(idx + 2).start()
            stage(idx).wait()

        plsc.subcore_barrier()
        write(idx).start()

    for idx in range(_HPC - _NBUF + 2, _HPC):
        write(idx).wait()


@functools.cache
def _make_materialize():
    mesh = plsc.VectorSubcoreMesh(core_axis_name="c", subcore_axis_name="s")
    return pl.kernel(
        _materialize_body,
        mesh=mesh,
        out_type=jax.ShapeDtypeStruct((_N_HEADS, _Q, _K), jnp.float32),
        scratch_types=(
            [pltpu.VMEM_SHARED((_N_SHIFTS, _DA), jnp.float32)] * _NBUF
            + [pltpu.SemaphoreType.DMA] * (_NBUF + 1)
        ),
    )


def kernel(query_length, key_length, offset, embeddings):
    del offset
    delta = (
        jnp.asarray(key_length, jnp.int32) - jnp.asarray(query_length, jnp.int32)
    ).reshape(1)
    shifts = _build_shifts(delta, embeddings.T)
    return _make_materialize()(shifts)
